# baseline (speedup 1.0000x reference)
import functools
import math

import jax
import jax.numpy as jnp
from jax import lax
from jax.experimental import pallas as pl
from jax.experimental.pallas import tpu as pltpu
from jax.experimental.pallas import tpu_sc as plsc

NUM_SAMPLES = 100000
NUM_FEATURES = 128
TEMP = 0.05
B = 1024
W = 2000
GRID = NUM_SAMPLES // W
LOG2E = math.log2(math.e)
LN2 = math.log(2.0)


def _lse_kernel(x_ref, g_ref, feat_ref, out_ref, xn_ref, s_ref, lbuf_ref):
    j = pl.program_id(0)

    @pl.when(j == 0)
    def _init():
        x = x_ref[...]
        norm = jnp.maximum(jnp.sqrt(jnp.sum(x * x, axis=1, keepdims=True)), 1e-12)
        xn_ref[...] = x * ((LOG2E / TEMP) / norm)
        s_ref[...] = jnp.zeros((B, 1), jnp.float32)

    @pl.when(j < GRID)
    def _mm():
        blk = feat_ref[...]
        l = lax.dot_general(xn_ref[...], blk, (((1,), (1,)), ((), ())),
                            preferred_element_type=jnp.float32)
        lbuf_ref[lax.rem(j, 2)] = l

    @pl.when(j > 0)
    def _consume():
        lp = lbuf_ref[lax.rem(j + 1, 2)]
        s_ref[...] += jnp.sum(jnp.exp2(lp), axis=1, keepdims=True)

    @pl.when(j == GRID)
    def _fin():
        t = jnp.sum(xn_ref[...] * g_ref[...], axis=1, keepdims=True)
        lse_minus_tgt = (jnp.log2(s_ref[...]) - t) * LN2
        out_ref[...] = jnp.sum(lse_minus_tgt, axis=(0, 1), keepdims=True) * (1.0 / B)


@jax.jit
def _run(x, feats, tgt):
    info = plsc.get_sparse_core_info()
    nw = info.num_cores * info.num_subcores
    bpw = B // nw
    mesh = plsc.VectorSubcoreMesh(core_axis_name="c", subcore_axis_name="s")

    @functools.partial(
        pl.kernel, mesh=mesh,
        out_type=jax.ShapeDtypeStruct((B, NUM_FEATURES), jnp.float32),
        scratch_types=[
            pltpu.VMEM((bpw,), jnp.int32),
            pltpu.VMEM((bpw, NUM_FEATURES), jnp.float32),
            pltpu.SemaphoreType.DMA,
        ],
    )
    def _sc_gather(table_hbm, idx_hbm, out_hbm, idx_v, rows_v, sem):
        wid = lax.axis_index("s") * info.num_cores + lax.axis_index("c")
        base = wid * bpw
        pltpu.sync_copy(idx_hbm.at[pl.ds(base, bpw)], idx_v)
        pltpu.async_copy(table_hbm.at[idx_v], rows_v, sem).wait()
        pltpu.sync_copy(rows_v, out_hbm.at[pl.ds(base, bpw)])

    g = _sc_gather(feats, tgt)

    out = pl.pallas_call(
        _lse_kernel,
        grid=(GRID + 1,),
        in_specs=[
            pl.BlockSpec((B, NUM_FEATURES), lambda j: (0, 0)),
            pl.BlockSpec((B, NUM_FEATURES), lambda j: (0, 0)),
            pl.BlockSpec((W, NUM_FEATURES), lambda j: (jnp.minimum(j, GRID - 1), 0)),
        ],
        out_specs=pl.BlockSpec((1, 1), lambda j: (0, 0)),
        out_shape=jax.ShapeDtypeStruct((1, 1), jnp.float32),
        scratch_shapes=[
            pltpu.VMEM((B, NUM_FEATURES), jnp.float32),
            pltpu.VMEM((B, 1), jnp.float32),
            pltpu.VMEM((2, B, W), jnp.float32),
        ],
    )(x, g, feats)
    return out[0, 0]


def kernel(inputs, features, targets, cam_ids):
    tgt = targets.astype(jnp.int32)
    return _run(inputs, features, tgt)


# speedup vs baseline: 1.2864x; 1.2864x over previous
import functools
import math

import jax
import jax.numpy as jnp
from jax import lax
from jax.experimental import pallas as pl
from jax.experimental.pallas import tpu as pltpu
from jax.experimental.pallas import tpu_sc as plsc

NUM_SAMPLES = 100000
NUM_FEATURES = 128
TEMP = 0.05
B = 1024
W = 2048
GRID = 48
TAIL = NUM_SAMPLES - GRID * W
LOG2E = math.log2(math.e)
LN2 = math.log(2.0)


def _lse_kernel(x_ref, g_ref, tail_ref, feat_ref, out_ref,
                xn_ref, acc_ref, st_ref):
    j = pl.program_id(0)

    @pl.when(j == 0)
    def _init():
        x = x_ref[...]
        norm = jnp.maximum(jnp.sqrt(jnp.sum(x * x, axis=1, keepdims=True)), 1e-12)
        xn = x * ((LOG2E / TEMP) / norm)
        xn_ref[...] = xn
        acc_ref[...] = jnp.zeros((B, NUM_FEATURES), jnp.float32)
        lt = lax.dot_general(xn, tail_ref[...], (((1,), (1,)), ((), ())),
                             preferred_element_type=jnp.float32)
        st_ref[...] = jnp.sum(jnp.exp2(lt), axis=1, keepdims=True)

    blk = feat_ref[...]
    l = lax.dot_general(xn_ref[...], blk, (((1,), (1,)), ((), ())),
                        preferred_element_type=jnp.float32)
    e = jnp.exp2(l.astype(jnp.bfloat16))
    e = e[:, :1024] + e[:, 1024:]
    e = e[:, :512] + e[:, 512:]
    e = e[:, :256] + e[:, 256:]
    e = e[:, :128] + e[:, 128:]
    acc_ref[...] += e.astype(jnp.float32)

    @pl.when(j == GRID - 1)
    def _fin():
        s_row = jnp.sum(acc_ref[...], axis=1, keepdims=True) + st_ref[...]
        t = jnp.sum(xn_ref[...] * g_ref[...], axis=1, keepdims=True)
        lse_minus_tgt = (jnp.log2(s_row) - t) * LN2
        out_ref[...] = jnp.sum(lse_minus_tgt, axis=(0, 1), keepdims=True) * (1.0 / B)


@jax.jit
def _run(x, feats, tgt):
    info = plsc.get_sparse_core_info()
    nw = info.num_cores * info.num_subcores
    bpw = B // nw
    mesh = plsc.VectorSubcoreMesh(core_axis_name="c", subcore_axis_name="s")

    @functools.partial(
        pl.kernel, mesh=mesh,
        out_type=jax.ShapeDtypeStruct((B, NUM_FEATURES), jnp.float32),
        scratch_types=[
            pltpu.VMEM((bpw,), jnp.int32),
            pltpu.VMEM((bpw, NUM_FEATURES), jnp.float32),
            pltpu.SemaphoreType.DMA,
        ],
    )
    def _sc_gather(table_hbm, idx_hbm, out_hbm, idx_v, rows_v, sem):
        wid = lax.axis_index("s") * info.num_cores + lax.axis_index("c")
        base = wid * bpw
        pltpu.sync_copy(idx_hbm.at[pl.ds(base, bpw)], idx_v)
        pltpu.async_copy(table_hbm.at[idx_v], rows_v, sem).wait()
        pltpu.sync_copy(rows_v, out_hbm.at[pl.ds(base, bpw)])

    g = _sc_gather(feats, tgt)

    tail = lax.slice(feats, (GRID * W, 0), (NUM_SAMPLES, NUM_FEATURES))
    out = pl.pallas_call(
        _lse_kernel,
        grid=(GRID,),
        in_specs=[
            pl.BlockSpec((B, NUM_FEATURES), lambda j: (0, 0)),
            pl.BlockSpec((B, NUM_FEATURES), lambda j: (0, 0)),
            pl.BlockSpec((TAIL, NUM_FEATURES), lambda j: (0, 0)),
            pl.BlockSpec((W, NUM_FEATURES), lambda j: (j, 0)),
        ],
        out_specs=pl.BlockSpec((1, 1), lambda j: (0, 0)),
        out_shape=jax.ShapeDtypeStruct((1, 1), jnp.float32),
        scratch_shapes=[
            pltpu.VMEM((B, NUM_FEATURES), jnp.float32),
            pltpu.VMEM((B, NUM_FEATURES), jnp.float32),
            pltpu.VMEM((B, 1), jnp.float32),
        ],
    )(x, g, tail, feats)
    return out[0, 0]


def kernel(inputs, features, targets, cam_ids):
    tgt = targets.astype(jnp.int32)
    return _run(inputs, features, tgt)


# speedup vs baseline: 1.6753x; 1.3023x over previous
import functools
import math

import jax
import jax.numpy as jnp
from jax import lax
from jax.experimental import pallas as pl
from jax.experimental.pallas import tpu as pltpu
from jax.experimental.pallas import tpu_sc as plsc

NUM_SAMPLES = 100000
NUM_FEATURES = 128
TEMP = 0.05
B = 1024
W = 4000
GRID = NUM_SAMPLES // W
LOG2E = math.log2(math.e)
LN2 = math.log(2.0)


def _lse_kernel(x_ref, g_ref, feat_ref, out_ref, xn_ref, s_ref):
    j = pl.program_id(0)

    @pl.when(j == 0)
    def _init():
        x = x_ref[...]
        norm = jnp.maximum(jnp.sqrt(jnp.sum(x * x, axis=1, keepdims=True)), 1e-12)
        xn_ref[...] = x * ((LOG2E / TEMP) / norm)
        s_ref[...] = jnp.zeros((B, 1), jnp.float32)

    xn = xn_ref[...]
    blk = feat_ref[...]
    l = lax.dot_general(xn, blk, (((1,), (1,)), ((), ())),
                        preferred_element_type=jnp.float32)
    s_ref[...] += jnp.sum(jnp.exp2(l), axis=1, keepdims=True)

    @pl.when(j == GRID - 1)
    def _fin():
        t = jnp.sum(xn * g_ref[...], axis=1, keepdims=True)
        lse_minus_tgt = (jnp.log2(s_ref[...]) - t) * LN2
        out_ref[...] = jnp.sum(lse_minus_tgt, axis=(0, 1), keepdims=True) * (1.0 / B)


@jax.jit
def _run(x, feats, tgt):
    info = plsc.get_sparse_core_info()
    nw = info.num_cores * info.num_subcores
    bpw = B // nw
    mesh = plsc.VectorSubcoreMesh(core_axis_name="c", subcore_axis_name="s")

    @functools.partial(
        pl.kernel, mesh=mesh,
        out_type=jax.ShapeDtypeStruct((B, NUM_FEATURES), jnp.float32),
        scratch_types=[
            pltpu.VMEM((bpw,), jnp.int32),
            pltpu.VMEM((bpw, NUM_FEATURES), jnp.float32),
            pltpu.SemaphoreType.DMA,
        ],
    )
    def _sc_gather(table_hbm, idx_hbm, out_hbm, idx_v, rows_v, sem):
        wid = lax.axis_index("s") * info.num_cores + lax.axis_index("c")
        base = wid * bpw
        pltpu.sync_copy(idx_hbm.at[pl.ds(base, bpw)], idx_v)
        pltpu.async_copy(table_hbm.at[idx_v], rows_v, sem).wait()
        pltpu.sync_copy(rows_v, out_hbm.at[pl.ds(base, bpw)])

    g = _sc_gather(feats, tgt)

    out = pl.pallas_call(
        _lse_kernel,
        grid=(GRID,),
        in_specs=[
            pl.BlockSpec((B, NUM_FEATURES), lambda j: (0, 0)),
            pl.BlockSpec((B, NUM_FEATURES), lambda j: (0, 0)),
            pl.BlockSpec((W, NUM_FEATURES), lambda j: (j, 0)),
        ],
        out_specs=pl.BlockSpec((1, 1), lambda j: (0, 0)),
        out_shape=jax.ShapeDtypeStruct((1, 1), jnp.float32),
        scratch_shapes=[
            pltpu.VMEM((B, NUM_FEATURES), jnp.float32),
            pltpu.VMEM((B, 1), jnp.float32),
        ],
    )(x, g, feats)
    return out[0, 0]


def kernel(inputs, features, targets, cam_ids):
    tgt = targets.astype(jnp.int32)
    return _run(inputs, features, tgt)


# speedup vs baseline: 1.7072x; 1.0190x over previous
import functools
import math

import jax
import jax.numpy as jnp
from jax import lax
from jax.experimental import pallas as pl
from jax.experimental.pallas import tpu as pltpu
from jax.experimental.pallas import tpu_sc as plsc

NUM_SAMPLES = 100000
NUM_FEATURES = 128
TEMP = 0.05
B = 1024
W = 5000
GRID = NUM_SAMPLES // W
LOG2E = math.log2(math.e)
LN2 = math.log(2.0)


def _lse_kernel(x_ref, g_ref, feat_ref, out_ref, xn_ref, s_ref):
    j = pl.program_id(0)

    @pl.when(j == 0)
    def _init():
        x = x_ref[...]
        norm = jnp.maximum(jnp.sqrt(jnp.sum(x * x, axis=1, keepdims=True)), 1e-12)
        xn_ref[...] = x * ((LOG2E / TEMP) / norm)
        s_ref[...] = jnp.zeros((B, 1), jnp.float32)

    xn = xn_ref[...]
    blk = feat_ref[...]
    l = lax.dot_general(xn, blk, (((1,), (1,)), ((), ())),
                        preferred_element_type=jnp.float32)
    s_ref[...] += jnp.sum(jnp.exp2(l), axis=1, keepdims=True)

    @pl.when(j == GRID - 1)
    def _fin():
        t = jnp.sum(xn * g_ref[...], axis=1, keepdims=True)
        lse_minus_tgt = (jnp.log2(s_ref[...]) - t) * LN2
        out_ref[...] = jnp.sum(lse_minus_tgt, axis=(0, 1), keepdims=True) * (1.0 / B)


@jax.jit
def _run(x, feats, tgt):
    info = plsc.get_sparse_core_info()
    nw = info.num_cores * info.num_subcores
    bpw = B // nw
    mesh = plsc.VectorSubcoreMesh(core_axis_name="c", subcore_axis_name="s")

    @functools.partial(
        pl.kernel, mesh=mesh,
        out_type=jax.ShapeDtypeStruct((B, NUM_FEATURES), jnp.float32),
        scratch_types=[
            pltpu.VMEM((bpw,), jnp.int32),
            pltpu.VMEM((bpw, NUM_FEATURES), jnp.float32),
            pltpu.SemaphoreType.DMA,
        ],
    )
    def _sc_gather(table_hbm, idx_hbm, out_hbm, idx_v, rows_v, sem):
        wid = lax.axis_index("s") * info.num_cores + lax.axis_index("c")
        base = wid * bpw
        pltpu.sync_copy(idx_hbm.at[pl.ds(base, bpw)], idx_v)
        pltpu.async_copy(table_hbm.at[idx_v], rows_v, sem).wait()
        pltpu.sync_copy(rows_v, out_hbm.at[pl.ds(base, bpw)])

    g = _sc_gather(feats, tgt)

    out = pl.pallas_call(
        _lse_kernel,
        grid=(GRID,),
        in_specs=[
            pl.BlockSpec((B, NUM_FEATURES), lambda j: (0, 0)),
            pl.BlockSpec((B, NUM_FEATURES), lambda j: (0, 0)),
            pl.BlockSpec((W, NUM_FEATURES), lambda j: (j, 0)),
        ],
        out_specs=pl.BlockSpec((1, 1), lambda j: (0, 0)),
        out_shape=jax.ShapeDtypeStruct((1, 1), jnp.float32),
        scratch_shapes=[
            pltpu.VMEM((B, NUM_FEATURES), jnp.float32),
            pltpu.VMEM((B, 1), jnp.float32),
        ],
    )(x, g, feats)
    return out[0, 0]


def kernel(inputs, features, targets, cam_ids):
    tgt = targets.astype(jnp.int32)
    return _run(inputs, features, tgt)


# speedup vs baseline: 1.7873x; 1.0470x over previous
import functools
import math

import jax
import jax.numpy as jnp
from jax import lax
from jax.experimental import pallas as pl
from jax.experimental.pallas import tpu as pltpu
from jax.experimental.pallas import tpu_sc as plsc

NUM_SAMPLES = 100000
NUM_FEATURES = 128
TEMP = 0.05
B = 1024
W = 10000
GRID = NUM_SAMPLES // W
LOG2E = math.log2(math.e)
LN2 = math.log(2.0)


def _lse_kernel(x_ref, g_ref, feat_ref, out_ref, xn_ref, s_ref):
    j = pl.program_id(0)

    @pl.when(j == 0)
    def _init():
        x = x_ref[...]
        norm = jnp.maximum(jnp.sqrt(jnp.sum(x * x, axis=1, keepdims=True)), 1e-12)
        xn_ref[...] = x * ((LOG2E / TEMP) / norm)
        s_ref[...] = jnp.zeros((B, 1), jnp.float32)

    xn = xn_ref[...]
    blk = feat_ref[...]
    l = lax.dot_general(xn, blk, (((1,), (1,)), ((), ())),
                        preferred_element_type=jnp.float32)
    s_ref[...] += jnp.sum(jnp.exp2(l), axis=1, keepdims=True)

    @pl.when(j == GRID - 1)
    def _fin():
        t = jnp.sum(xn * g_ref[...], axis=1, keepdims=True)
        lse_minus_tgt = (jnp.log2(s_ref[...]) - t) * LN2
        out_ref[...] = jnp.sum(lse_minus_tgt, axis=(0, 1), keepdims=True) * (1.0 / B)


@jax.jit
def _run(x, feats, tgt):
    info = plsc.get_sparse_core_info()
    nw = info.num_cores * info.num_subcores
    bpw = B // nw
    mesh = plsc.VectorSubcoreMesh(core_axis_name="c", subcore_axis_name="s")

    @functools.partial(
        pl.kernel, mesh=mesh,
        out_type=jax.ShapeDtypeStruct((B, NUM_FEATURES), jnp.float32),
        scratch_types=[
            pltpu.VMEM((bpw,), jnp.int32),
            pltpu.VMEM((bpw, NUM_FEATURES), jnp.float32),
            pltpu.SemaphoreType.DMA,
        ],
    )
    def _sc_gather(table_hbm, idx_hbm, out_hbm, idx_v, rows_v, sem):
        wid = lax.axis_index("s") * info.num_cores + lax.axis_index("c")
        base = wid * bpw
        pltpu.sync_copy(idx_hbm.at[pl.ds(base, bpw)], idx_v)
        pltpu.async_copy(table_hbm.at[idx_v], rows_v, sem).wait()
        pltpu.sync_copy(rows_v, out_hbm.at[pl.ds(base, bpw)])

    g = _sc_gather(feats, tgt)

    out = pl.pallas_call(
        _lse_kernel,
        grid=(GRID,),
        in_specs=[
            pl.BlockSpec((B, NUM_FEATURES), lambda j: (0, 0)),
            pl.BlockSpec((B, NUM_FEATURES), lambda j: (0, 0)),
            pl.BlockSpec((W, NUM_FEATURES), lambda j: (j, 0)),
        ],
        out_specs=pl.BlockSpec((1, 1), lambda j: (0, 0)),
        out_shape=jax.ShapeDtypeStruct((1, 1), jnp.float32),
        scratch_shapes=[
            pltpu.VMEM((B, NUM_FEATURES), jnp.float32),
            pltpu.VMEM((B, 1), jnp.float32),
        ],
    )(x, g, feats)
    return out[0, 0]


def kernel(inputs, features, targets, cam_ids):
    tgt = targets.astype(jnp.int32)
    return _run(inputs, features, tgt)


# speedup vs baseline: 1.8134x; 1.0146x over previous
import functools
import math

import jax
import jax.numpy as jnp
from jax import lax
from jax.experimental import pallas as pl
from jax.experimental.pallas import tpu as pltpu
from jax.experimental.pallas import tpu_sc as plsc

NUM_SAMPLES = 100000
NUM_FEATURES = 128
TEMP = 0.05
B = 1024
W = 10000
GRID = NUM_SAMPLES // W
LOG2E = math.log2(math.e)
LN2 = math.log(2.0)


def _lse_kernel(x_ref, feat_ref, out_ref, xn_ref, s_ref):
    j = pl.program_id(0)

    @pl.when(j == 0)
    def _init():
        x = x_ref[...]
        norm = jnp.maximum(jnp.sqrt(jnp.sum(x * x, axis=1, keepdims=True)), 1e-12)
        xn_ref[...] = x * ((LOG2E / TEMP) / norm)
        s_ref[...] = jnp.zeros((B, 1), jnp.float32)

    xn = xn_ref[...]
    blk = feat_ref[...]
    l = lax.dot_general(xn, blk, (((1,), (1,)), ((), ())),
                        preferred_element_type=jnp.float32)
    s_ref[...] += jnp.sum(jnp.exp2(l), axis=1, keepdims=True)

    @pl.when(j == GRID - 1)
    def _fin():
        out_ref[...] = jnp.sum(jnp.log2(s_ref[...]), axis=(0, 1), keepdims=True)


def _tgt_kernel(x_ref, g_ref, out_ref):
    x = x_ref[...]
    norm = jnp.maximum(jnp.sqrt(jnp.sum(x * x, axis=1, keepdims=True)), 1e-12)
    xn = x * ((LOG2E / TEMP) / norm)
    t = jnp.sum(xn * g_ref[...], axis=1, keepdims=True)
    out_ref[...] = jnp.sum(t, axis=(0, 1), keepdims=True)


@jax.jit
def _run(x, feats, tgt):
    info = plsc.get_sparse_core_info()
    nw = info.num_cores * info.num_subcores
    bpw = B // nw
    mesh = plsc.VectorSubcoreMesh(core_axis_name="c", subcore_axis_name="s")

    @functools.partial(
        pl.kernel, mesh=mesh,
        out_type=jax.ShapeDtypeStruct((B, NUM_FEATURES), jnp.float32),
        scratch_types=[
            pltpu.VMEM((bpw,), jnp.int32),
            pltpu.VMEM((bpw, NUM_FEATURES), jnp.float32),
            pltpu.SemaphoreType.DMA,
        ],
    )
    def _sc_gather(table_hbm, idx_hbm, out_hbm, idx_v, rows_v, sem):
        wid = lax.axis_index("s") * info.num_cores + lax.axis_index("c")
        base = wid * bpw
        pltpu.sync_copy(idx_hbm.at[pl.ds(base, bpw)], idx_v)
        pltpu.async_copy(table_hbm.at[idx_v], rows_v, sem).wait()
        pltpu.sync_copy(rows_v, out_hbm.at[pl.ds(base, bpw)])

    g = _sc_gather(feats, tgt)

    lse_sum = pl.pallas_call(
        _lse_kernel,
        grid=(GRID,),
        in_specs=[
            pl.BlockSpec((B, NUM_FEATURES), lambda j: (0, 0)),
            pl.BlockSpec((W, NUM_FEATURES), lambda j: (j, 0)),
        ],
        out_specs=pl.BlockSpec((1, 1), lambda j: (0, 0)),
        out_shape=jax.ShapeDtypeStruct((1, 1), jnp.float32),
        scratch_shapes=[
            pltpu.VMEM((B, NUM_FEATURES), jnp.float32),
            pltpu.VMEM((B, 1), jnp.float32),
        ],
    )(x, feats)

    t_sum = pl.pallas_call(
        _tgt_kernel,
        out_shape=jax.ShapeDtypeStruct((1, 1), jnp.float32),
    )(x, g)

    return (lse_sum[0, 0] - t_sum[0, 0]) * (LN2 / B)


def kernel(inputs, features, targets, cam_ids):
    tgt = targets.astype(jnp.int32)
    return _run(inputs, features, tgt)


# speedup vs baseline: 1.8416x; 1.0156x over previous
import functools
import math

import jax
import jax.numpy as jnp
from jax import lax
from jax.experimental import pallas as pl
from jax.experimental.pallas import tpu as pltpu
from jax.experimental.pallas import tpu_sc as plsc

NUM_SAMPLES = 100000
NUM_FEATURES = 128
TEMP = 0.05
B = 1024
W = 10000
GRID = NUM_SAMPLES // W
LOG2E = math.log2(math.e)
LN2 = math.log(2.0)


def _lse_kernel(x_ref, feat_ref, out_ref, xn_ref, s_ref):
    j = pl.program_id(0)

    @pl.when(j == 0)
    def _init():
        x = x_ref[...]
        norm = jnp.maximum(jnp.sqrt(jnp.sum(x * x, axis=1, keepdims=True)), 1e-12)
        xn_ref[...] = x * ((LOG2E / TEMP) / norm)
        s_ref[...] = jnp.zeros((B, 1), jnp.float32)

    xn = xn_ref[...]
    blk = feat_ref[...]
    l = lax.dot_general(xn, blk, (((1,), (1,)), ((), ())),
                        preferred_element_type=jnp.float32)
    s_ref[...] += jnp.sum(jnp.exp2(l), axis=1, keepdims=True)

    @pl.when(j == GRID - 1)
    def _fin():
        out_ref[...] = jnp.sum(jnp.log2(s_ref[...]), axis=(0, 1), keepdims=True)


def _tgt_kernel(x_ref, g_ref, lse_ref, out_ref):
    x = x_ref[...]
    norm = jnp.maximum(jnp.sqrt(jnp.sum(x * x, axis=1, keepdims=True)), 1e-12)
    xn = x * ((LOG2E / TEMP) / norm)
    t = jnp.sum(xn * g_ref[...], axis=1, keepdims=True)
    t_sum = jnp.sum(t, axis=(0, 1), keepdims=True)
    out_ref[...] = (lse_ref[...] - t_sum) * (LN2 / B)


@jax.jit
def _run(x, feats, tgt):
    info = plsc.get_sparse_core_info()
    nw = info.num_cores * info.num_subcores
    bpw = B // nw
    mesh = plsc.VectorSubcoreMesh(core_axis_name="c", subcore_axis_name="s")

    @functools.partial(
        pl.kernel, mesh=mesh,
        out_type=jax.ShapeDtypeStruct((B, NUM_FEATURES), jnp.float32),
        scratch_types=[
            pltpu.VMEM((bpw,), jnp.int32),
            pltpu.VMEM((bpw, NUM_FEATURES), jnp.float32),
            pltpu.SemaphoreType.DMA,
        ],
    )
    def _sc_gather(table_hbm, idx_hbm, out_hbm, idx_v, rows_v, sem):
        wid = lax.axis_index("s") * info.num_cores + lax.axis_index("c")
        base = wid * bpw
        pltpu.sync_copy(idx_hbm.at[pl.ds(base, bpw)], idx_v)
        pltpu.async_copy(table_hbm.at[idx_v], rows_v, sem).wait()
        pltpu.sync_copy(rows_v, out_hbm.at[pl.ds(base, bpw)])

    g = _sc_gather(feats, tgt)

    lse_sum = pl.pallas_call(
        _lse_kernel,
        grid=(GRID,),
        in_specs=[
            pl.BlockSpec((B, NUM_FEATURES), lambda j: (0, 0)),
            pl.BlockSpec((W, NUM_FEATURES), lambda j: (j, 0)),
        ],
        out_specs=pl.BlockSpec((1, 1), lambda j: (0, 0)),
        out_shape=jax.ShapeDtypeStruct((1, 1), jnp.float32),
        scratch_shapes=[
            pltpu.VMEM((B, NUM_FEATURES), jnp.float32),
            pltpu.VMEM((B, 1), jnp.float32),
        ],
    )(x, feats)

    out = pl.pallas_call(
        _tgt_kernel,
        out_shape=jax.ShapeDtypeStruct((1, 1), jnp.float32),
    )(x, g, lse_sum)
    return out[0, 0]


def kernel(inputs, features, targets, cam_ids):
    tgt = targets.astype(jnp.int32)
    return _run(inputs, features, tgt)


# speedup vs baseline: 1.8546x; 1.0070x over previous
import functools
import math

import jax
import jax.numpy as jnp
from jax import lax
from jax.experimental import pallas as pl
from jax.experimental.pallas import tpu as pltpu
from jax.experimental.pallas import tpu_sc as plsc

NUM_SAMPLES = 100000
NUM_FEATURES = 128
TEMP = 0.05
B = 1024
W = 20000
GRID = NUM_SAMPLES // W
LOG2E = math.log2(math.e)
LN2 = math.log(2.0)


def _lse_kernel(x_ref, feat_ref, out_ref, xn_ref, s_ref):
    j = pl.program_id(0)

    @pl.when(j == 0)
    def _init():
        x = x_ref[...]
        norm = jnp.maximum(jnp.sqrt(jnp.sum(x * x, axis=1, keepdims=True)), 1e-12)
        xn_ref[...] = x * ((LOG2E / TEMP) / norm)
        s_ref[...] = jnp.zeros((B, 1), jnp.float32)

    xn = xn_ref[...]
    blk = feat_ref[...]
    l = lax.dot_general(xn, blk, (((1,), (1,)), ((), ())),
                        preferred_element_type=jnp.float32)
    s_ref[...] += jnp.sum(jnp.exp2(l), axis=1, keepdims=True)

    @pl.when(j == GRID - 1)
    def _fin():
        out_ref[...] = jnp.sum(jnp.log2(s_ref[...]), axis=(0, 1), keepdims=True)


def _tgt_kernel(x_ref, g_ref, lse_ref, out_ref):
    x = x_ref[...]
    norm = jnp.maximum(jnp.sqrt(jnp.sum(x * x, axis=1, keepdims=True)), 1e-12)
    xn = x * ((LOG2E / TEMP) / norm)
    t = jnp.sum(xn * g_ref[...], axis=1, keepdims=True)
    t_sum = jnp.sum(t, axis=(0, 1), keepdims=True)
    out_ref[...] = (lse_ref[...] - t_sum) * (LN2 / B)


@jax.jit
def _run(x, feats, tgt):
    info = plsc.get_sparse_core_info()
    nw = info.num_cores * info.num_subcores
    bpw = B // nw
    mesh = plsc.VectorSubcoreMesh(core_axis_name="c", subcore_axis_name="s")

    @functools.partial(
        pl.kernel, mesh=mesh,
        out_type=jax.ShapeDtypeStruct((B, NUM_FEATURES), jnp.float32),
        scratch_types=[
            pltpu.VMEM((bpw,), jnp.int32),
            pltpu.VMEM((bpw, NUM_FEATURES), jnp.float32),
            pltpu.SemaphoreType.DMA,
        ],
    )
    def _sc_gather(table_hbm, idx_hbm, out_hbm, idx_v, rows_v, sem):
        wid = lax.axis_index("s") * info.num_cores + lax.axis_index("c")
        base = wid * bpw
        pltpu.sync_copy(idx_hbm.at[pl.ds(base, bpw)], idx_v)
        pltpu.async_copy(table_hbm.at[idx_v], rows_v, sem).wait()
        pltpu.sync_copy(rows_v, out_hbm.at[pl.ds(base, bpw)])

    g = _sc_gather(feats, tgt)

    lse_sum = pl.pallas_call(
        _lse_kernel,
        grid=(GRID,),
        in_specs=[
            pl.BlockSpec((B, NUM_FEATURES), lambda j: (0, 0)),
            pl.BlockSpec((W, NUM_FEATURES), lambda j: (j, 0)),
        ],
        out_specs=pl.BlockSpec((1, 1), lambda j: (0, 0)),
        out_shape=jax.ShapeDtypeStruct((1, 1), jnp.float32),
        scratch_shapes=[
            pltpu.VMEM((B, NUM_FEATURES), jnp.float32),
            pltpu.VMEM((B, 1), jnp.float32),
        ],
        compiler_params=pltpu.CompilerParams(vmem_limit_bytes=120 * 1024 * 1024),
    )(x, feats)

    out = pl.pallas_call(
        _tgt_kernel,
        out_shape=jax.ShapeDtypeStruct((1, 1), jnp.float32),
    )(x, g, lse_sum)
    return out[0, 0]


def kernel(inputs, features, targets, cam_ids):
    tgt = targets.astype(jnp.int32)
    return _run(inputs, features, tgt)
